# Initial kernel scaffold; baseline (speedup 1.0000x reference)
import jax
import jax.numpy as jnp
from jax.experimental import pallas as pl


def kernel(X, hyperedge_index, S_features, W_w, W_b):
    raise NotImplementedError("write your pallas kernel here")


# speedup vs baseline: 7.2058x; 7.2058x over previous
import dataclasses
import functools

import jax
import jax.numpy as jnp
from jax import lax
from jax.experimental import pallas as pl
from jax.experimental.pallas import tpu as pltpu
from jax.experimental.pallas import tpu_sc as plsc

C = 128
NC, NS = 2, 16
NW = NC * NS
CHUNK = 128
LANES = 16


def _sc_mesh():
    return plsc.VectorSubcoreMesh(core_axis_name="c", subcore_axis_name="s")


def _sc_compiler_params():
    cp = pltpu.CompilerParams()
    if "needs_layout_passes" in pltpu.CompilerParams.__dataclass_fields__:
        cp = dataclasses.replace(cp, needs_layout_passes=False)
    return cp


def _padded_rows(n_rows):
    per_tile = ((n_rows + NS * CHUNK - 1) // (NS * CHUNK)) * CHUNK
    return per_tile, NS * per_tile


def _zero_1d(ref, total):
    @pl.loop(0, total, step=LANES)
    def _(i):
        ref[pl.ds(i, LANES)] = jnp.zeros((LANES,), jnp.float32)


def _make_sc_stats(n_rows, nnz):
    nchunk = nnz // CHUNK
    rpt, n_pad = _padded_rows(n_rows)

    @functools.partial(
        pl.kernel,
        out_type=(
            jax.ShapeDtypeStruct((n_pad,), jnp.float32),
            jax.ShapeDtypeStruct((n_pad,), jnp.float32),
            jax.ShapeDtypeStruct((n_pad,), jnp.float32),
        ),
        mesh=_sc_mesh(),
        compiler_params=_sc_compiler_params(),
        scratch_types=[
            pltpu.VMEM((CHUNK,), jnp.int32),
            pltpu.VMEM((CHUNK,), jnp.int32),
            pltpu.VMEM((n_pad,), jnp.float32),
            pltpu.VMEM((n_pad,), jnp.float32),
            pltpu.VMEM((n_pad,), jnp.float32),
            pltpu.VMEM((rpt,), jnp.float32),
            pltpu.VMEM((rpt,), jnp.float32),
            pltpu.VMEM_SHARED((NS * n_pad,), jnp.float32),
            pltpu.VMEM_SHARED((n_pad,), jnp.float32),
        ],
    )
    def stats(v_hbm, e_hbm, dv_hbm, cnt_hbm, de_hbm,
              vb, eb, h1, h2, dvfull, racc, rtmp, stage, dvred):
        cid = lax.axis_index("c")
        sid = lax.axis_index("s")
        ones = jnp.ones((LANES,), jnp.float32)

        def reduce_private(src_ref, out_hbm_ref, red_ref):
            pltpu.sync_copy(src_ref, stage.at[pl.ds(sid * n_pad, n_pad)])
            plsc.subcore_barrier()
            pltpu.sync_copy(stage.at[pl.ds(sid * rpt, rpt)], racc)

            @pl.loop(1, NS)
            def _(t):
                pltpu.sync_copy(
                    stage.at[pl.ds(t * n_pad + sid * rpt, rpt)], rtmp)

                @pl.loop(0, rpt, step=LANES)
                def _(k):
                    racc[pl.ds(k, LANES)] = (
                        racc[pl.ds(k, LANES)] + rtmp[pl.ds(k, LANES)])

            @pl.when(cid == 0)
            def _():
                pltpu.sync_copy(racc, out_hbm_ref.at[pl.ds(sid * rpt, rpt)])
            if red_ref is not None:
                pltpu.sync_copy(racc, red_ref.at[pl.ds(sid * rpt, rpt)])
            plsc.subcore_barrier()

        _zero_1d(h1, n_pad)

        @pl.loop(sid, nchunk, step=NS)
        def _(c):
            pltpu.sync_copy(v_hbm.at[pl.ds(c * CHUNK, CHUNK)], vb)

            @pl.loop(0, CHUNK, step=LANES)
            def _(j):
                plsc.addupdate_scatter(h1, [vb[pl.ds(j, LANES)]], ones)

        reduce_private(h1, dv_hbm, dvred)
        pltpu.sync_copy(dvred, dvfull)
        plsc.subcore_barrier()

        _zero_1d(h1, n_pad)
        _zero_1d(h2, n_pad)

        @pl.loop(sid, nchunk, step=NS)
        def _(c):
            pltpu.sync_copy(v_hbm.at[pl.ds(c * CHUNK, CHUNK)], vb)
            pltpu.sync_copy(e_hbm.at[pl.ds(c * CHUNK, CHUNK)], eb)

            @pl.loop(0, CHUNK, step=LANES)
            def _(j):
                ei = eb[pl.ds(j, LANES)]
                plsc.addupdate_scatter(h1, [ei], ones)
                dvv = plsc.load_gather(dvfull, [vb[pl.ds(j, LANES)]])
                plsc.addupdate_scatter(h2, [ei], dvv)

        reduce_private(h1, cnt_hbm, None)
        reduce_private(h2, de_hbm, None)

    return stats


def _make_sc_segsum(n_rows, nnz):
    nchunk = nnz // CHUNK
    rpt, n_pad = _padded_rows(n_rows)

    @functools.partial(
        pl.kernel,
        out_type=jax.ShapeDtypeStruct((NC * n_pad, C), jnp.float32),
        mesh=_sc_mesh(),
        compiler_params=_sc_compiler_params(),
        scratch_types=[
            pltpu.VMEM((CHUNK,), jnp.int32),
            pltpu.VMEM((CHUNK,), jnp.int32),
            pltpu.VMEM((CHUNK, C), jnp.float32),
            pltpu.VMEM_SHARED((n_pad, C), jnp.float32),
            pltpu.SemaphoreType.DMA,
        ],
    )
    def segsum(table_hbm, src_hbm, dst_hbm, out_hbm, src_v, dst_v, rows_v,
               acc_s, sem):
        cid = lax.axis_index("c")
        sid = lax.axis_index("s")
        wid = sid * NC + cid

        @pl.loop(0, CHUNK)
        def _(i):
            @pl.loop(0, C, step=LANES)
            def _(j):
                rows_v[i, pl.ds(j, LANES)] = jnp.zeros((LANES,), jnp.float32)

        @pl.loop(0, rpt // CHUNK)
        def _(z):
            pltpu.sync_copy(
                rows_v, acc_s.at[pl.ds(sid * rpt + z * CHUNK, CHUNK)])

        plsc.subcore_barrier()

        @pl.loop(wid, nchunk, step=NW)
        def _(c):
            base = c * CHUNK
            pltpu.sync_copy(src_hbm.at[pl.ds(base, CHUNK)], src_v)
            pltpu.sync_copy(dst_hbm.at[pl.ds(base, CHUNK)], dst_v)
            pltpu.async_copy(table_hbm.at[src_v], rows_v, sem).wait()
            pltpu.sync_copy(rows_v, acc_s.at[dst_v], add=True)

        plsc.subcore_barrier()
        pltpu.sync_copy(
            acc_s.at[pl.ds(sid * rpt, rpt)],
            out_hbm.at[pl.ds(cid * n_pad + sid * rpt, rpt)],
        )

    return segsum


def _tc_linear(X, W_w, W_b):
    n = X.shape[0]

    def body(x_ref, w_ref, b_ref, out_ref):
        out_ref[...] = lax.dot_general(
            x_ref[...], w_ref[...], (((1,), (1,)), ((), ())),
            preferred_element_type=jnp.float32,
        ) + b_ref[...][None, :]

    return pl.pallas_call(
        body, out_shape=jax.ShapeDtypeStruct((n, C), jnp.float32)
    )(X, W_w, W_b)


def _tc_normalize(a0, a1, cnt, de_sum):
    n = a0.shape[0]

    def body(a0_ref, a1_ref, cnt_ref, de_ref, y_ref):
        sums = a0_ref[...] + a1_ref[...]
        cnt = cnt_ref[...]
        mean = jnp.where(cnt > 0, sums / jnp.maximum(cnt, 1.0), 0.0)
        de = de_ref[...] / (cnt + 1.0)
        de_r = jnp.where(cnt > 0, lax.rsqrt(jnp.maximum(de, 1e-30)), 1.0)
        y_ref[...] = de_r * mean

    return pl.pallas_call(
        body, out_shape=jax.ShapeDtypeStruct((n, C), jnp.float32)
    )(a0, a1, cnt, de_sum)


def _tc_finalize(b0, b1, dv):
    n = b0.shape[0]

    def body(b0_ref, b1_ref, dv_ref, out_ref):
        xn = b0_ref[...] + b1_ref[...]
        d = dv_ref[...]
        dv_r = jnp.where(d > 0, lax.rsqrt(jnp.maximum(d, 1.0)), 0.0)
        out_ref[...] = jnp.maximum(dv_r * xn, 0.0)

    return pl.pallas_call(
        body, out_shape=jax.ShapeDtypeStruct((n, C), jnp.float32)
    )(b0, b1, dv)


def kernel(X, hyperedge_index, S_features, W_w, W_b):
    del S_features
    n = X.shape[0]
    nnz = hyperedge_index.shape[1]
    V = hyperedge_index[0]
    E = hyperedge_index[1]
    _, n_pad = _padded_rows(n)

    dv, cnt, de = _make_sc_stats(n, nnz)(V, E)
    xl = _tc_linear(X, W_w, W_b)

    acc_a = _make_sc_segsum(n, nnz)(xl, V, E)
    y = _tc_normalize(acc_a[:n], acc_a[n_pad:n_pad + n],
                      cnt[:n, None], de[:n, None])

    acc_b = _make_sc_segsum(n, nnz)(y, E, V)
    return _tc_finalize(acc_b[:n], acc_b[n_pad:n_pad + n], dv[:n, None])
